# baseline (speedup 1.0000x reference)
import functools

import jax
import jax.numpy as jnp
from jax import lax
from jax.experimental import pallas as pl
from jax.experimental.pallas import tpu as pltpu
from jax.experimental.pallas import tpu_sc as plsc

N = 10000
D = 128
H = 128
HH = H // 2
E = 320000
L = 3
NIMG = 16

NC = 2
NS = 16
KE = 128
CH = 158
EPAD = NS * CH * KE
ZROW = N
TAB_ROWS = N + 16
AGG_ROWS = 10240
RPT = AGG_ROWS // NS
RCH = 128

PH_SYM = (0, 40, 80, 120, CH)
PH_DIR = (0, 80, CH)

BLK = 2000


def _zero_zbuf(zbuf):
    def zrow(i, _):
        def zcol(j, _):
            zbuf[i, pl.ds(j * 32, 32)] = jnp.zeros((32,), jnp.bfloat16)
            return 0
        return lax.fori_loop(0, HH // 32, zcol, 0)
    lax.fori_loop(0, RCH, zrow, 0)


def _mk_pack(col, rows, qr):
    def ext(r, _):
        qr[r, pl.ds(0, 32)] = plsc.pack(
            rows[r, pl.ds(col, 16)], rows[r, pl.ds(col + 16, 16)],
            format=plsc.PackFormat.INTERLEAVED)
        qr[r, pl.ds(32, 32)] = plsc.pack(
            rows[r, pl.ds(col + 32, 16)], rows[r, pl.ds(col + 48, 16)],
            format=plsc.PackFormat.INTERLEAVED)
        return 0
    return lambda: lax.fori_loop(0, KE, ext, 0, unroll=8)


def _zero_agg(zbuf, agg_sh, base):
    def zslice(k, _):
        pltpu.sync_copy(zbuf, agg_sh.at[pl.ds(base + k * RCH, RCH)])
        return 0
    lax.fori_loop(0, RPT // RCH, zslice, 0)


def _copy_out(agg_sh, out, bstage, fstage, fprev, base, col, first):
    def orow(r, _):
        a, b = plsc.unpack(bstage[r, pl.ds(0, 32)],
                           format=plsc.PackFormat.INTERLEAVED)
        c, d = plsc.unpack(bstage[r, pl.ds(32, 32)],
                           format=plsc.PackFormat.INTERLEAVED)
        if not first:
            a = a + fprev[r, pl.ds(0, 16)]
            b = b + fprev[r, pl.ds(16, 16)]
            c = c + fprev[r, pl.ds(32, 16)]
            d = d + fprev[r, pl.ds(48, 16)]
        fstage[r, pl.ds(0, 16)] = a
        fstage[r, pl.ds(16, 16)] = b
        fstage[r, pl.ds(32, 16)] = c
        fstage[r, pl.ds(48, 16)] = d
        return 0

    def oslice(k, _):
        pltpu.sync_copy(agg_sh.at[pl.ds(base + k * RCH, RCH)], bstage)
        if not first:
            pltpu.sync_copy(
                out.at[pl.ds(base + k * RCH, RCH), pl.ds(col, HH)], fprev)
        lax.fori_loop(0, RCH, orow, 0, unroll=4)
        pltpu.sync_copy(
            fstage, out.at[pl.ds(base + k * RCH, RCH), pl.ds(col, HH)])
        return 0
    lax.fori_loop(0, RPT // RCH, oslice, 0)


def _halves(c, one_half):
    @pl.when(c == 0)
    def _():
        one_half(0)

    @pl.when(c == 1)
    def _():
        one_half(1)


def _sym_body(tab, pa, pb, out,
              av, bv, rows0, rows1, q0, q1, zbuf, bstage, fstage, fprev, agg_sh,
              sg0, sg1, ss0, ss1):
    c = lax.axis_index("c")
    s = lax.axis_index("s")
    _zero_zbuf(zbuf)
    pltpu.sync_copy(pa.at[s], av)
    pltpu.sync_copy(pb.at[s], bv)
    base = s * RPT

    def wait_gather(buf, sem):
        pltpu.make_async_copy(tab.at[pl.ds(0, KE)], buf, sem).wait()

    def wait_scatter(qr, sem):
        pltpu.make_async_copy(qr, agg_sh.at[pl.ds(0, KE)], sem).wait()

    def one_half(h):
        col = h * HH
        ex0 = _mk_pack(col, rows0, q0)
        ex1 = _mk_pack(col, rows1, q1)

        for ph in range(len(PH_SYM) - 1):
            lo, hi = PH_SYM[ph], PH_SYM[ph + 1]
            _zero_agg(zbuf, agg_sh, base)
            plsc.subcore_barrier()

            pltpu.async_copy(tab.at[av.at[lo]], rows0, sg0)

            def chunk(j, _):
                wait_gather(rows0, sg0)
                pltpu.async_copy(tab.at[bv.at[j]], rows1, sg1)

                @pl.when(j > lo)
                def _():
                    wait_scatter(q0, ss0)
                ex0()
                pltpu.async_copy(q0, agg_sh.at[bv.at[j]], ss0, add=True)

                wait_gather(rows1, sg1)

                @pl.when(j < hi - 1)
                def _():
                    pltpu.async_copy(tab.at[av.at[j + 1]], rows0, sg0)

                @pl.when(j > lo)
                def _():
                    wait_scatter(q1, ss1)
                ex1()
                pltpu.async_copy(q1, agg_sh.at[av.at[j]], ss1, add=True)
                return 0
            lax.fori_loop(lo, hi, chunk, 0)
            wait_scatter(q0, ss0)
            wait_scatter(q1, ss1)
            plsc.subcore_barrier()
            _copy_out(agg_sh, out, bstage, fstage, fprev, base, col,
                      ph == 0)

    _halves(c, one_half)


def _dir_body(tab, pg, psc, out,
              gv, sv, rows0, rows1, q0, q1, zbuf, bstage, fstage, fprev, agg_sh,
              sg0, sg1, ss0, ss1):
    c = lax.axis_index("c")
    s = lax.axis_index("s")
    _zero_zbuf(zbuf)
    pltpu.sync_copy(pg.at[s], gv)
    pltpu.sync_copy(psc.at[s], sv)
    base = s * RPT

    def wait_gather(buf, sem):
        pltpu.make_async_copy(tab.at[pl.ds(0, KE)], buf, sem).wait()

    def wait_scatter(qr, sem):
        pltpu.make_async_copy(qr, agg_sh.at[pl.ds(0, KE)], sem).wait()

    def one_half(h):
        col = h * HH
        ex0 = _mk_pack(col, rows0, q0)
        ex1 = _mk_pack(col, rows1, q1)

        for ph in range(len(PH_DIR) - 1):
            lo, hi = PH_DIR[ph], PH_DIR[ph + 1]
            _zero_agg(zbuf, agg_sh, base)
            plsc.subcore_barrier()

            pltpu.async_copy(tab.at[gv.at[lo]], rows0, sg0)

            def pair(p, _):
                j0 = 2 * p
                j1 = 2 * p + 1
                wait_gather(rows0, sg0)
                pltpu.async_copy(tab.at[gv.at[j1]], rows1, sg1)

                @pl.when(p > lo // 2)
                def _():
                    wait_scatter(q0, ss0)
                ex0()
                pltpu.async_copy(q0, agg_sh.at[sv.at[j0]], ss0, add=True)

                wait_gather(rows1, sg1)

                @pl.when(p < hi // 2 - 1)
                def _():
                    pltpu.async_copy(tab.at[gv.at[j1 + 1]], rows0, sg0)

                @pl.when(p > lo // 2)
                def _():
                    wait_scatter(q1, ss1)
                ex1()
                pltpu.async_copy(q1, agg_sh.at[sv.at[j1]], ss1, add=True)
                return 0
            lax.fori_loop(lo // 2, hi // 2, pair, 0)
            wait_scatter(q0, ss0)
            wait_scatter(q1, ss1)
            plsc.subcore_barrier()
            _copy_out(agg_sh, out, bstage, fstage, fprev, base, col,
                      ph == 0)

    _halves(c, one_half)


def _sc_kernel(body):
    return pl.kernel(
        body,
        out_type=jax.ShapeDtypeStruct((AGG_ROWS, H), jnp.float32),
        mesh=plsc.VectorSubcoreMesh(core_axis_name="c", subcore_axis_name="s"),
        compiler_params=pltpu.CompilerParams(
            use_tc_tiling_on_sc=False, needs_layout_passes=False),
        scratch_types=[
            pltpu.VMEM((CH, KE), jnp.int32),
            pltpu.VMEM((CH, KE), jnp.int32),
            pltpu.VMEM((KE, H), jnp.float32),
            pltpu.VMEM((KE, H), jnp.float32),
            pltpu.VMEM((KE, HH), jnp.bfloat16),
            pltpu.VMEM((KE, HH), jnp.bfloat16),
            pltpu.VMEM((RCH, HH), jnp.bfloat16),
            pltpu.VMEM((RCH, HH), jnp.bfloat16),
            pltpu.VMEM((RCH, HH), jnp.float32),
            pltpu.VMEM((RCH, HH), jnp.float32),
            pltpu.VMEM_SHARED((AGG_ROWS, HH), jnp.bfloat16),
            pltpu.SemaphoreType.DMA,
            pltpu.SemaphoreType.DMA,
            pltpu.SemaphoreType.DMA,
            pltpu.SemaphoreType.DMA,
        ],
    )


@functools.cache
def _sym_pass():
    return _sc_kernel(_sym_body)


@functools.cache
def _dir_pass():
    return _sc_kernel(_dir_body)


def _linear_body(x_ref, w_ref, b_ref, o_ref):
    o_ref[...] = (
        jnp.dot(x_ref[...], w_ref[...], preferred_element_type=jnp.float32)
        + b_ref[...]
    )


def _linear(x, w, b):
    return pl.pallas_call(
        _linear_body,
        grid=(N // BLK,),
        in_specs=[
            pl.BlockSpec((BLK, D), lambda i: (i, 0)),
            pl.BlockSpec((D, H), lambda i: (0, 0)),
            pl.BlockSpec((1, H), lambda i: (0, 0)),
        ],
        out_specs=pl.BlockSpec((BLK, H), lambda i: (i, 0)),
        out_shape=jax.ShapeDtypeStruct((N, H), jnp.float32),
    )(x, w, b.reshape(1, H))


def _layer_body(ns_ref, np_ref, r1_ref, r2_ref, w1_ref, w2_ref, b_ref, o_ref):
    ns = ns_ref[...]
    agg = np_ref[...] + r1_ref[...] + r2_ref[...]
    z = (
        jnp.dot(ns, w1_ref[...], preferred_element_type=jnp.float32)
        + jnp.dot(agg, w2_ref[...], preferred_element_type=jnp.float32)
        + b_ref[...]
    )
    o_ref[...] = ns + z * jax.nn.sigmoid(z)


def _layer(ns, nparts, rp1, rp2, w1, w2, b):
    return pl.pallas_call(
        _layer_body,
        grid=(N // BLK,),
        in_specs=[
            pl.BlockSpec((BLK, H), lambda i: (i, 0)),
            pl.BlockSpec((BLK, H), lambda i: (i, 0)),
            pl.BlockSpec((BLK, H), lambda i: (i, 0)),
            pl.BlockSpec((BLK, H), lambda i: (i, 0)),
            pl.BlockSpec((H, H), lambda i: (0, 0)),
            pl.BlockSpec((H, H), lambda i: (0, 0)),
            pl.BlockSpec((1, H), lambda i: (0, 0)),
        ],
        out_specs=pl.BlockSpec((BLK, H), lambda i: (i, 0)),
        out_shape=jax.ShapeDtypeStruct((N, H), jnp.float32),
    )(ns, nparts, rp1, rp2, w1, w2, b.reshape(1, H))


def _final_body(ns_ref, img_ref, g_ref, b_ref, out_ref, gs_ref, sums, counts):
    i = pl.program_id(0)

    @pl.when(i == 0)
    def _():
        sums[...] = jnp.zeros_like(sums)
        counts[...] = jnp.zeros_like(counts)

    x = ns_ref[...]
    mu = jnp.mean(x, axis=-1, keepdims=True)
    var = jnp.mean((x - mu) ** 2, axis=-1, keepdims=True)
    y = (x - mu) * lax.rsqrt(var + 1e-5) * g_ref[...] + b_ref[...]
    out_ref[...] = y

    img = img_ref[0, 0, :]
    oh = (
        lax.broadcasted_iota(jnp.int32, (NIMG, BLK), 0) == img[None, :]
    ).astype(jnp.float32)
    sums[...] += jnp.dot(oh, y, preferred_element_type=jnp.float32)
    counts[...] += jnp.broadcast_to(
        jnp.sum(oh, axis=1, keepdims=True), (NIMG, H)
    )

    @pl.when(i == N // BLK - 1)
    def _():
        gs_ref[...] = sums[...] / jnp.maximum(counts[...], 1.0)


def _final(ns, img3, g, b):
    return pl.pallas_call(
        _final_body,
        grid=(N // BLK,),
        in_specs=[
            pl.BlockSpec((BLK, H), lambda i: (i, 0)),
            pl.BlockSpec((1, 1, BLK), lambda i: (i, 0, 0)),
            pl.BlockSpec((1, H), lambda i: (0, 0)),
            pl.BlockSpec((1, H), lambda i: (0, 0)),
        ],
        out_specs=[
            pl.BlockSpec((BLK, H), lambda i: (i, 0)),
            pl.BlockSpec((NIMG, H), lambda i: (0, 0)),
        ],
        out_shape=[
            jax.ShapeDtypeStruct((N, H), jnp.float32),
            jax.ShapeDtypeStruct((NIMG, H), jnp.float32),
        ],
        scratch_shapes=[
            pltpu.VMEM((NIMG, H), jnp.float32),
            pltpu.VMEM((NIMG, H), jnp.float32),
        ],
    )(ns, img3, g.reshape(1, H), b.reshape(1, H))


def _padtab(x):
    pad = jnp.zeros((TAB_ROWS - N, H), jnp.float32)
    return jnp.concatenate([x, pad])


def kernel(node_feats, rel_feats, triples, obj_to_img,
           W_node_in, b_node_in, W_rel_in, b_rel_in,
           proj_W, proj_b, ln_node_g, ln_node_b, ln_rel_g, ln_rel_b):
    subj = triples[:, 0]
    rel = triples[:, 1]
    obj = triples[:, 2]
    pad = jnp.full((EPAD - E,), ZROW, jnp.int32)

    def pidx(x):
        return jnp.concatenate([x, pad]).reshape(NS, CH, KE)

    subj_p = pidx(subj)
    obj_p = pidx(obj)
    rel_p = pidx(rel)

    ns = _linear(node_feats, W_node_in, b_node_in)
    rs = _linear(rel_feats, W_rel_in, b_rel_in)

    sym = _sym_pass()
    dr = _dir_pass()
    rs_tab = _padtab(rs)
    rp1 = dr(rs_tab, rel_p, obj_p)[:N]
    rp2 = dr(rs_tab, rel_p, subj_p)[:N]
    for i in range(L):
        nparts = sym(_padtab(ns), subj_p, obj_p)[:N]
        ns = _layer(ns, nparts, rp1, rp2, proj_W[i, :H], proj_W[i, H:],
                    proj_b[i])

    img3 = obj_to_img.reshape(N // BLK, 1, BLK)
    ns_out, gs = _final(ns, img3, ln_node_g, ln_node_b)
    return ns_out, gs


# speedup vs baseline: 1.1645x; 1.1645x over previous
import functools

import jax
import jax.numpy as jnp
from jax import lax
from jax.experimental import pallas as pl
from jax.experimental.pallas import tpu as pltpu
from jax.experimental.pallas import tpu_sc as plsc

N = 10000
D = 128
H = 128
HH = H // 2
E = 320000
L = 3
NIMG = 16

NC = 2
NS = 16
KE = 128
CH = 158
EPAD = NS * CH * KE
ZROW = N
TAB_ROWS = N + 16
AGG_ROWS = 10240
RPT = AGG_ROWS // NS
RCH = 128

PH_SYM = (0, 40, 80, 120, CH)
PH_DIR = (0, 80, CH)

BLK = 2000


def _zero_zbuf(zbuf):
    def zrow(i, _):
        def zcol(j, _):
            zbuf[i, pl.ds(j * 32, 32)] = jnp.zeros((32,), jnp.bfloat16)
            return 0
        return lax.fori_loop(0, HH // 32, zcol, 0)
    lax.fori_loop(0, RCH, zrow, 0)


def _mk_pack(col, rows, qr):
    def ext(r, _):
        qr[r, pl.ds(0, 32)] = plsc.pack(
            rows[r, pl.ds(col, 16)], rows[r, pl.ds(col + 16, 16)],
            format=plsc.PackFormat.INTERLEAVED)
        qr[r, pl.ds(32, 32)] = plsc.pack(
            rows[r, pl.ds(col + 32, 16)], rows[r, pl.ds(col + 48, 16)],
            format=plsc.PackFormat.INTERLEAVED)
        return 0
    return lambda: lax.fori_loop(0, KE, ext, 0, unroll=8)


def _zero_agg(zbuf, agg_sh, base):
    def zslice(k, _):
        pltpu.sync_copy(zbuf, agg_sh.at[pl.ds(base + k * RCH, RCH)])
        return 0
    lax.fori_loop(0, RPT // RCH, zslice, 0)


def _copy_out(agg_sh, out, bstage, fstage, fprev, base, col, first):
    def orow(r, _):
        a, b = plsc.unpack(bstage[r, pl.ds(0, 32)],
                           format=plsc.PackFormat.INTERLEAVED)
        c, d = plsc.unpack(bstage[r, pl.ds(32, 32)],
                           format=plsc.PackFormat.INTERLEAVED)
        if not first:
            a = a + fprev[r, pl.ds(0, 16)]
            b = b + fprev[r, pl.ds(16, 16)]
            c = c + fprev[r, pl.ds(32, 16)]
            d = d + fprev[r, pl.ds(48, 16)]
        fstage[r, pl.ds(0, 16)] = a
        fstage[r, pl.ds(16, 16)] = b
        fstage[r, pl.ds(32, 16)] = c
        fstage[r, pl.ds(48, 16)] = d
        return 0

    def oslice(k, _):
        pltpu.sync_copy(agg_sh.at[pl.ds(base + k * RCH, RCH)], bstage)
        if not first:
            pltpu.sync_copy(
                out.at[pl.ds(base + k * RCH, RCH), pl.ds(col, HH)], fprev)
        lax.fori_loop(0, RCH, orow, 0, unroll=4)
        pltpu.sync_copy(
            fstage, out.at[pl.ds(base + k * RCH, RCH), pl.ds(col, HH)])
        return 0
    lax.fori_loop(0, RPT // RCH, oslice, 0)


def _halves(c, one_half):
    @pl.when(c == 0)
    def _():
        one_half(0)

    @pl.when(c == 1)
    def _():
        one_half(1)


def _sym_body(tab, pa, pb, out,
              av, bv, rows0, rows1, q0, q1, zbuf, bstage, fstage, fprev, agg_sh,
              sg0, sg1, ss0, ss1):
    c = lax.axis_index("c")
    s = lax.axis_index("s")
    _zero_zbuf(zbuf)
    pltpu.sync_copy(pa.at[s], av)
    pltpu.sync_copy(pb.at[s], bv)
    base = s * RPT

    def wait_gather(buf, sem):
        pltpu.make_async_copy(tab.at[pl.ds(0, KE)], buf, sem).wait()

    def wait_scatter(qr, sem):
        pltpu.make_async_copy(qr, agg_sh.at[pl.ds(0, KE)], sem).wait()

    def one_half(h):
        col = h * HH
        ex0 = _mk_pack(col, rows0, q0)
        ex1 = _mk_pack(col, rows1, q1)

        for ph in range(len(PH_SYM) - 1):
            lo, hi = PH_SYM[ph], PH_SYM[ph + 1]
            _zero_agg(zbuf, agg_sh, base)
            plsc.subcore_barrier()

            pltpu.async_copy(tab.at[av.at[lo]], rows0, sg0)
            pltpu.async_copy(tab.at[bv.at[lo]], rows1, sg1)

            def chunk(j, _):
                wait_gather(rows0, sg0)

                @pl.when(j > lo)
                def _():
                    wait_scatter(q0, ss0)
                ex0()

                @pl.when(j < hi - 1)
                def _():
                    pltpu.async_copy(tab.at[av.at[j + 1]], rows0, sg0)
                pltpu.async_copy(q0, agg_sh.at[bv.at[j]], ss0, add=True)

                wait_gather(rows1, sg1)

                @pl.when(j > lo)
                def _():
                    wait_scatter(q1, ss1)
                ex1()

                @pl.when(j < hi - 1)
                def _():
                    pltpu.async_copy(tab.at[bv.at[j + 1]], rows1, sg1)
                pltpu.async_copy(q1, agg_sh.at[av.at[j]], ss1, add=True)
                return 0
            lax.fori_loop(lo, hi, chunk, 0)
            wait_scatter(q0, ss0)
            wait_scatter(q1, ss1)
            plsc.subcore_barrier()
            _copy_out(agg_sh, out, bstage, fstage, fprev, base, col,
                      ph == 0)

    _halves(c, one_half)


def _rel_body(tab, pg, po, ps, out,
              gv, ov, sv, rows0, rows1, q0, q1, zbuf, bstage, fstage, fprev,
              agg_sh, sg0, sg1, ss0, ss1):
    c = lax.axis_index("c")
    s = lax.axis_index("s")
    _zero_zbuf(zbuf)
    base = s * RPT

    def wait_gather(buf, sem):
        pltpu.make_async_copy(tab.at[pl.ds(0, KE)], buf, sem).wait()

    def wait_scatter(qr, sem):
        pltpu.make_async_copy(qr, agg_sh.at[pl.ds(0, KE)], sem).wait()

    def one_half(h):
        col = h * HH
        ex0 = _mk_pack(col, rows0, q0)
        ex1 = _mk_pack(col, rows1, q1)

        for ph in range(len(PH_DIR) - 1):
            lo, hi = PH_DIR[ph], PH_DIR[ph + 1]
            psz = hi - lo
            pltpu.sync_copy(pg.at[s, pl.ds(lo, psz)], gv.at[pl.ds(0, psz)])
            pltpu.sync_copy(po.at[s, pl.ds(lo, psz)], ov.at[pl.ds(0, psz)])
            pltpu.sync_copy(ps.at[s, pl.ds(lo, psz)], sv.at[pl.ds(0, psz)])
            _zero_agg(zbuf, agg_sh, base)
            plsc.subcore_barrier()

            pltpu.async_copy(tab.at[gv.at[0]], rows0, sg0)
            pltpu.async_copy(tab.at[gv.at[1]], rows1, sg1)

            def pair(p, _):
                j0 = 2 * p
                j1 = 2 * p + 1
                wait_gather(rows0, sg0)

                @pl.when(p > 0)
                def _():
                    wait_scatter(q0, ss0)
                    wait_scatter(q0, ss0)
                ex0()

                @pl.when(j0 + 2 < psz)
                def _():
                    pltpu.async_copy(tab.at[gv.at[j0 + 2]], rows0, sg0)
                pltpu.async_copy(q0, agg_sh.at[ov.at[j0]], ss0, add=True)
                pltpu.async_copy(q0, agg_sh.at[sv.at[j0]], ss0, add=True)

                wait_gather(rows1, sg1)

                @pl.when(p > 0)
                def _():
                    wait_scatter(q1, ss1)
                    wait_scatter(q1, ss1)
                ex1()

                @pl.when(j1 + 2 < psz)
                def _():
                    pltpu.async_copy(tab.at[gv.at[j1 + 2]], rows1, sg1)
                pltpu.async_copy(q1, agg_sh.at[ov.at[j1]], ss1, add=True)
                pltpu.async_copy(q1, agg_sh.at[sv.at[j1]], ss1, add=True)
                return 0
            lax.fori_loop(0, psz // 2, pair, 0)
            wait_scatter(q0, ss0)
            wait_scatter(q0, ss0)
            wait_scatter(q1, ss1)
            wait_scatter(q1, ss1)
            plsc.subcore_barrier()
            _copy_out(agg_sh, out, bstage, fstage, fprev, base, col,
                      ph == 0)

    _halves(c, one_half)


def _sc_kernel(body, n_idx, n_sem, idx_rows=CH):
    return pl.kernel(
        body,
        out_type=jax.ShapeDtypeStruct((AGG_ROWS, H), jnp.float32),
        mesh=plsc.VectorSubcoreMesh(core_axis_name="c", subcore_axis_name="s"),
        compiler_params=pltpu.CompilerParams(
            use_tc_tiling_on_sc=False, needs_layout_passes=False),
        scratch_types=(
            [pltpu.VMEM((idx_rows, KE), jnp.int32)] * n_idx
            + [
                pltpu.VMEM((KE, H), jnp.float32),
                pltpu.VMEM((KE, H), jnp.float32),
                pltpu.VMEM((KE, HH), jnp.bfloat16),
                pltpu.VMEM((KE, HH), jnp.bfloat16),
                pltpu.VMEM((RCH, HH), jnp.bfloat16),
                pltpu.VMEM((RCH, HH), jnp.bfloat16),
                pltpu.VMEM((RCH, HH), jnp.float32),
                pltpu.VMEM((RCH, HH), jnp.float32),
                pltpu.VMEM_SHARED((AGG_ROWS, HH), jnp.bfloat16),
            ]
            + [pltpu.SemaphoreType.DMA] * n_sem
        ),
    )


@functools.cache
def _sym_pass():
    return _sc_kernel(_sym_body, 2, 4)


@functools.cache
def _rel_pass():
    return _sc_kernel(_rel_body, 3, 4, idx_rows=80)


def _linear_body(x_ref, w_ref, b_ref, o_ref):
    o_ref[...] = (
        jnp.dot(x_ref[...], w_ref[...], preferred_element_type=jnp.float32)
        + b_ref[...]
    )


def _linear(x, w, b):
    return pl.pallas_call(
        _linear_body,
        grid=(N // BLK,),
        in_specs=[
            pl.BlockSpec((BLK, D), lambda i: (i, 0)),
            pl.BlockSpec((D, H), lambda i: (0, 0)),
            pl.BlockSpec((1, H), lambda i: (0, 0)),
        ],
        out_specs=pl.BlockSpec((BLK, H), lambda i: (i, 0)),
        out_shape=jax.ShapeDtypeStruct((N, H), jnp.float32),
    )(x, w, b.reshape(1, H))


def _layer_body(ns_ref, np_ref, rp_ref, w1_ref, w2_ref, b_ref, o_ref):
    ns = ns_ref[...]
    agg = np_ref[...] + rp_ref[...]
    z = (
        jnp.dot(ns, w1_ref[...], preferred_element_type=jnp.float32)
        + jnp.dot(agg, w2_ref[...], preferred_element_type=jnp.float32)
        + b_ref[...]
    )
    o_ref[...] = ns + z * jax.nn.sigmoid(z)


def _layer(ns, nparts, rparts, w1, w2, b):
    return pl.pallas_call(
        _layer_body,
        grid=(N // BLK,),
        in_specs=[
            pl.BlockSpec((BLK, H), lambda i: (i, 0)),
            pl.BlockSpec((BLK, H), lambda i: (i, 0)),
            pl.BlockSpec((BLK, H), lambda i: (i, 0)),
            pl.BlockSpec((H, H), lambda i: (0, 0)),
            pl.BlockSpec((H, H), lambda i: (0, 0)),
            pl.BlockSpec((1, H), lambda i: (0, 0)),
        ],
        out_specs=pl.BlockSpec((BLK, H), lambda i: (i, 0)),
        out_shape=jax.ShapeDtypeStruct((N, H), jnp.float32),
    )(ns, nparts, rparts, w1, w2, b.reshape(1, H))


def _final_body(ns_ref, img_ref, g_ref, b_ref, out_ref, gs_ref, sums, counts):
    i = pl.program_id(0)

    @pl.when(i == 0)
    def _():
        sums[...] = jnp.zeros_like(sums)
        counts[...] = jnp.zeros_like(counts)

    x = ns_ref[...]
    mu = jnp.mean(x, axis=-1, keepdims=True)
    var = jnp.mean((x - mu) ** 2, axis=-1, keepdims=True)
    y = (x - mu) * lax.rsqrt(var + 1e-5) * g_ref[...] + b_ref[...]
    out_ref[...] = y

    img = img_ref[0, 0, :]
    oh = (
        lax.broadcasted_iota(jnp.int32, (NIMG, BLK), 0) == img[None, :]
    ).astype(jnp.float32)
    sums[...] += jnp.dot(oh, y, preferred_element_type=jnp.float32)
    counts[...] += jnp.broadcast_to(
        jnp.sum(oh, axis=1, keepdims=True), (NIMG, H)
    )

    @pl.when(i == N // BLK - 1)
    def _():
        gs_ref[...] = sums[...] / jnp.maximum(counts[...], 1.0)


def _final(ns, img3, g, b):
    return pl.pallas_call(
        _final_body,
        grid=(N // BLK,),
        in_specs=[
            pl.BlockSpec((BLK, H), lambda i: (i, 0)),
            pl.BlockSpec((1, 1, BLK), lambda i: (i, 0, 0)),
            pl.BlockSpec((1, H), lambda i: (0, 0)),
            pl.BlockSpec((1, H), lambda i: (0, 0)),
        ],
        out_specs=[
            pl.BlockSpec((BLK, H), lambda i: (i, 0)),
            pl.BlockSpec((NIMG, H), lambda i: (0, 0)),
        ],
        out_shape=[
            jax.ShapeDtypeStruct((N, H), jnp.float32),
            jax.ShapeDtypeStruct((NIMG, H), jnp.float32),
        ],
        scratch_shapes=[
            pltpu.VMEM((NIMG, H), jnp.float32),
            pltpu.VMEM((NIMG, H), jnp.float32),
        ],
    )(ns, img3, g.reshape(1, H), b.reshape(1, H))


def _padtab(x):
    pad = jnp.zeros((TAB_ROWS - N, H), jnp.float32)
    return jnp.concatenate([x, pad])


def kernel(node_feats, rel_feats, triples, obj_to_img,
           W_node_in, b_node_in, W_rel_in, b_rel_in,
           proj_W, proj_b, ln_node_g, ln_node_b, ln_rel_g, ln_rel_b):
    subj = triples[:, 0]
    rel = triples[:, 1]
    obj = triples[:, 2]
    pad = jnp.full((EPAD - E,), ZROW, jnp.int32)

    def pidx(x):
        return jnp.concatenate([x, pad]).reshape(NS, CH, KE)

    subj_p = pidx(subj)
    obj_p = pidx(obj)
    rel_p = pidx(rel)

    ns = _linear(node_feats, W_node_in, b_node_in)
    rs = _linear(rel_feats, W_rel_in, b_rel_in)

    sym = _sym_pass()
    rparts = _rel_pass()(_padtab(rs), rel_p, obj_p, subj_p)[:N]
    for i in range(L):
        nparts = sym(_padtab(ns), subj_p, obj_p)[:N]
        ns = _layer(ns, nparts, rparts, proj_W[i, :H], proj_W[i, H:],
                    proj_b[i])

    img3 = obj_to_img.reshape(N // BLK, 1, BLK)
    ns_out, gs = _final(ns, img3, ln_node_g, ln_node_b)
    return ns_out, gs
